# baseline (speedup 1.0000x reference)
import dataclasses
import functools

import jax
import jax.numpy as jnp
from jax import lax
from jax.experimental import pallas as pl
from jax.experimental.pallas import tpu as pltpu
from jax.experimental.pallas import tpu_sc as plsc

F = 26
V = 100000
D = 32
B = 16384
L = 16
NC = 2
NW = 32
PER_W = F * D // NW
Q = 4096
NQ = B // Q
VH = 50048

_mesh = plsc.VectorSubcoreMesh(core_axis_name="c", subcore_axis_name="s")

_cp = pltpu.CompilerParams(use_tc_tiling_on_sc=True)
if "needs_layout_passes" in pltpu.CompilerParams.__dataclass_fields__:
    _cp = dataclasses.replace(_cp, needs_layout_passes=False)


@functools.partial(
    pl.kernel,
    mesh=_mesh,
    out_type=jax.ShapeDtypeStruct((F * D, B), jnp.float32),
    scratch_types=[
        pltpu.VMEM((1, B), jnp.int32),
        pltpu.VMEM((1, V), jnp.float32),
        pltpu.VMEM((1, Q), jnp.float32),
        pltpu.VMEM((1, Q), jnp.float32),
        pltpu.SemaphoreType.DMA,
        pltpu.SemaphoreType.DMA,
        pltpu.SemaphoreType.DMA,
        pltpu.SemaphoreType.DMA,
    ],
    compiler_params=_cp,
)
def _field_gather(xt_hbm, t2_hbm, out_hbm, idx_v, vec_v, out_a, out_b,
                  vsem_a, vsem_b, wsem_a, wsem_b):
    wid = lax.axis_index("s") * NC + lax.axis_index("c")
    fd0 = wid * PER_W

    @pl.loop(0, PER_W)
    def _(k):
        fd = fd0 + k
        f = lax.shift_right_logical(fd, 5)
        VQ = 25088
        for (s0, sl, sem) in ((0, VQ, vsem_a), (VQ, VQ, vsem_b),
                              (2 * VQ, VQ, vsem_a), (3 * VQ, V - 3 * VQ,
                                                     vsem_b)):
            pltpu.async_copy(t2_hbm.at[pl.ds(fd, 1), pl.ds(s0, sl)],
                             vec_v.at[:, pl.ds(s0, sl)], sem)

        @pl.when((k == 0) | (lax.bitwise_and(fd, D - 1) == 0))
        def _():
            pltpu.sync_copy(xt_hbm.at[pl.ds(f, 1), :], idx_v)

        for (s0, sl, sem) in ((0, VQ, vsem_a), (VQ, VQ, vsem_b),
                              (2 * VQ, VQ, vsem_a), (3 * VQ, V - 3 * VQ,
                                                     vsem_b)):
            pltpu.make_async_copy(t2_hbm.at[pl.ds(0, 1), pl.ds(s0, sl)],
                                  vec_v.at[:, pl.ds(s0, sl)], sem).wait()

        for q in range(NQ):
            out_v, wsem = (out_a, wsem_a) if q % 2 == 0 else (out_b, wsem_b)
            if q < 2:
                @pl.when(k > 0)
                def _():
                    pltpu.make_async_copy(
                        out_v, out_hbm.at[pl.ds(0, 1), pl.ds(0, Q)],
                        wsem).wait()
            else:
                pltpu.make_async_copy(
                    out_v, out_hbm.at[pl.ds(0, 1), pl.ds(0, Q)],
                    wsem).wait()

            @plsc.parallel_loop(0, Q, step=L, unroll=8)
            def _(j):
                iv = idx_v[0, pl.ds(q * Q + j, L)]
                out_v[0, pl.ds(j, L)] = plsc.load_gather(vec_v.at[0], [iv])

            pltpu.async_copy(
                out_v, out_hbm.at[pl.ds(fd, 1), pl.ds(q * Q, Q)], wsem)

    pltpu.make_async_copy(out_a, out_hbm.at[pl.ds(0, 1), pl.ds(0, Q)],
                          wsem_a).wait()
    pltpu.make_async_copy(out_b, out_hbm.at[pl.ds(0, 1), pl.ds(0, Q)],
                          wsem_b).wait()


def kernel(X, tables):
    xt = X.T
    tt = jnp.transpose(tables, (0, 2, 1))
    t2 = tt.reshape(F * D, V)
    out_t = _field_gather(xt, t2)
    return out_t.T
